# baseline (speedup 1.0000x reference)
import functools

import jax
import jax.numpy as jnp
from jax import lax
from jax.experimental import pallas as pl
from jax.experimental.pallas import tpu as pltpu
from jax.experimental.pallas import tpu_sc as plsc

_NC, _NS, _L = 2, 16, 16
_NW = _NC * _NS
_S_PAD = 56
_CHUNK_OFFS = (0, 16, 32, 48, 64, 80, 84)


def _make_pooled(B, S, D, V):
    assert S == 50 and D == 100 and B % _NW == 0
    RPW = B // _NW
    PAD_IDX = V - 2
    mesh = plsc.VectorSubcoreMesh(core_axis_name="c", subcore_axis_name="s")

    @functools.partial(
        pl.kernel,
        out_type=jax.ShapeDtypeStruct((B, D), jnp.float32),
        mesh=mesh,
        scratch_types=[
            pltpu.VMEM((RPW, S), jnp.int32),
            pltpu.VMEM((RPW, S), jnp.int32),
            pltpu.VMEM((RPW, _S_PAD), jnp.int32),
            pltpu.VMEM((_S_PAD, D), jnp.float32),
            pltpu.VMEM((_S_PAD, D), jnp.float32),
            pltpu.VMEM((RPW, D), jnp.float32),
            pltpu.SemaphoreType.DMA,
            pltpu.SemaphoreType.DMA,
        ],
        compiler_params=pltpu.CompilerParams(
            needs_layout_passes=False, use_tc_tiling_on_sc=False),
    )
    def pooled(ids_hbm, mask_hbm, table_hbm, out_hbm,
               ids_v, mask_v, idx_v, buf0, buf1, out_v, sem0, sem1):
        wid = lax.axis_index("s") * _NC + lax.axis_index("c")
        base = wid * RPW
        pltpu.sync_copy(ids_hbm.at[pl.ds(base, RPW)], ids_v)
        pltpu.sync_copy(mask_hbm.at[pl.ds(base, RPW)], mask_v)

        lanes = lax.iota(jnp.int32, _L)
        pad_vec = jnp.full((_L,), PAD_IDX, jnp.int32)

        def prep(b, carry):
            idrow = ids_v.at[b]
            mrow = mask_v.at[b]
            orow = idx_v.at[b]
            for off in (0, 16, 32):
                m = mrow[pl.ds(off, _L)]
                orow[pl.ds(off, _L)] = jnp.where(m != 0, idrow[pl.ds(off, _L)], pad_vec)
            orow[pl.ds(40, _L)] = pad_vec
            m3 = mrow[pl.ds(34, _L)]
            orow[pl.ds(34, _L)] = jnp.where(m3 != 0, idrow[pl.ds(34, _L)], pad_vec)
            return carry

        lax.fori_loop(0, RPW, prep, jnp.int32(0))

        def gather(b, buf, sem):
            return pltpu.make_async_copy(table_hbm.at[idx_v.at[b]], buf, sem)

        def accum(b, buf):
            def step(s, accs):
                row = buf.at[s]
                return tuple(a + row[pl.ds(off, _L)] for a, off in zip(accs, _CHUNK_OFFS))

            accs = lax.fori_loop(
                0, _S_PAD, step,
                tuple(jnp.zeros((_L,), jnp.float32) for _ in _CHUNK_OFFS))
            mrow = mask_v.at[b]
            cnt = plsc.all_reduce_population_count(mrow[pl.ds(0, _L)] != 0)
            cnt = cnt + plsc.all_reduce_population_count(mrow[pl.ds(16, _L)] != 0)
            cnt = cnt + plsc.all_reduce_population_count(mrow[pl.ds(32, _L)] != 0)
            cnt = cnt + plsc.all_reduce_population_count(
                (mrow[pl.ds(34, _L)] != 0) & (lanes >= 14))
            scale = 1.0 / jnp.maximum(cnt.astype(jnp.float32), 1e-9)
            orow = out_v.at[b]
            for a, off in zip(accs, _CHUNK_OFFS):
                orow[pl.ds(off, _L)] = a * scale

        gather(0, buf0, sem0).start()

        def main(i, carry):
            b0 = 2 * i
            gather(b0 + 1, buf1, sem1).start()
            gather(b0, buf0, sem0).wait()
            accum(b0, buf0)

            @pl.when(i < RPW // 2 - 1)
            def _():
                gather(b0 + 2, buf0, sem0).start()

            gather(b0 + 1, buf1, sem1).wait()
            accum(b0 + 1, buf1)
            return carry

        lax.fori_loop(0, RPW // 2, main, jnp.int32(0))
        pltpu.sync_copy(out_v, out_hbm.at[pl.ds(base, RPW)])

    return pooled


@jax.jit
def _run(ids, msk, tbl):
    B, S = ids.shape
    V, D = tbl.shape
    return _make_pooled(B, S, D, V)(ids, msk, tbl)


def kernel(input_ids, attention_mask, embedding_table):
    return _run(input_ids.astype(jnp.int32),
                attention_mask.astype(jnp.int32),
                embedding_table.astype(jnp.float32))
